# baseline (speedup 1.0000x reference)
import functools
import jax
import jax.numpy as jnp
from jax import lax
from jax.experimental import pallas as pl
from jax.experimental.pallas import tpu as pltpu
from jax.experimental.pallas import tpu_sc as plsc

N = 16384
V = 1000

_info = plsc.get_sparse_core_info()
NC, NS, L = _info.num_cores, _info.num_subcores, _info.num_lanes
NW = NC * NS
ROWS_PER_W = N // NW
CHUNK = 16
NCHUNK = ROWS_PER_W // CHUNK
GROUPS = CHUNK // L

_mesh = plsc.VectorSubcoreMesh(core_axis_name="c", subcore_axis_name="s")


@functools.partial(
    pl.kernel,
    mesh=_mesh,
    out_type=jax.ShapeDtypeStruct((N, V), jnp.float32),
    scratch_types=[
        pltpu.VMEM((ROWS_PER_W,), jnp.int32),
        pltpu.VMEM((CHUNK, V), jnp.float32),
        pltpu.VMEM((CHUNK, V), jnp.float32),
        pltpu.VMEM((CHUNK, V), jnp.float32),
        pltpu.VMEM((CHUNK, V), jnp.float32),
        pltpu.SemaphoreType.DMA,
        pltpu.SemaphoreType.DMA,
        pltpu.SemaphoreType.DMA,
        pltpu.SemaphoreType.DMA,
    ],
    compiler_params=pltpu.CompilerParams(needs_layout_passes=False),
)
def _sc_onehot(x_hbm, out_hbm, idx_v, buf0, buf1, buf2, buf3, sem0, sem1, sem2, sem3):
    wid = lax.axis_index("s") * NC + lax.axis_index("c")
    base_row = wid * ROWS_PER_W

    pltpu.sync_copy(x_hbm.at[pl.ds(base_row, ROWS_PER_W)], idx_v)

    zeros16 = jnp.zeros((L,), jnp.float32)
    lane = lax.iota(jnp.int32, L)

    def _zero_row(r, carry):
        def _zero_k(k, c2):
            off = pl.multiple_of(k * L, L)
            buf0[r, pl.ds(off, L)] = zeros16
            buf1[r, pl.ds(off, L)] = zeros16
            buf2[r, pl.ds(off, L)] = zeros16
            buf3[r, pl.ds(off, L)] = zeros16
            return c2

        lax.fori_loop(0, V // L, _zero_k, 0)
        rows = jnp.full((L,), r, jnp.int32)
        tail = (V - L) + lane
        plsc.store_scatter(buf0, [rows, tail], zeros16)
        plsc.store_scatter(buf1, [rows, tail], zeros16)
        plsc.store_scatter(buf2, [rows, tail], zeros16)
        plsc.store_scatter(buf3, [rows, tail], zeros16)
        return carry

    lax.fori_loop(0, CHUNK, _zero_row, 0)

    bufs = (buf0, buf1, buf2, buf3)
    sems = (sem0, sem1, sem2, sem3)
    ones16 = jnp.ones((L,), jnp.float32)

    copies = [None] * NCHUNK
    positions = [None] * NCHUNK
    for c in range(NCHUNK):
        b = c % 4
        if c >= 4:
            copies[c - 4].wait()
            for rows, cols in positions[c - 4]:
                plsc.store_scatter(bufs[b], [rows, cols], zeros16)
        pos_list = []
        for g in range(GROUPS):
            xs = idx_v[pl.ds(c * CHUNK + g * L, L)]
            rows = g * L + lane
            plsc.store_scatter(bufs[b], [rows, xs], ones16)
            pos_list.append((rows, xs))
        positions[c] = pos_list
        row0 = base_row + c * CHUNK
        cp = pltpu.make_async_copy(
            bufs[b],
            out_hbm.at[pl.ds(row0, CHUNK), :],
            sems[b],
        )
        cp.start()
        copies[c] = cp
    for d in range(4):
        copies[NCHUNK - 4 + d].wait()


def kernel(X):
    return _sc_onehot(X)


# speedup vs baseline: 1.0031x; 1.0031x over previous
import functools
import jax
import jax.numpy as jnp
from jax import lax
from jax.experimental import pallas as pl
from jax.experimental.pallas import tpu as pltpu
from jax.experimental.pallas import tpu_sc as plsc

N = 16384
V = 1000

_info = plsc.get_sparse_core_info()
NC, NS, L = _info.num_cores, _info.num_subcores, _info.num_lanes
NW = NC * NS
ROWS_PER_W = N // NW
CHUNK = 16
NCHUNK = ROWS_PER_W // CHUNK
GROUPS = CHUNK // L

_mesh = plsc.VectorSubcoreMesh(core_axis_name="c", subcore_axis_name="s")


@functools.partial(
    pl.kernel,
    mesh=_mesh,
    out_type=jax.ShapeDtypeStruct((N, V), jnp.float32),
    scratch_types=[
        pltpu.VMEM((ROWS_PER_W,), jnp.int32),
        pltpu.VMEM((CHUNK, V), jnp.float32),
        pltpu.VMEM((CHUNK, V), jnp.float32),
        pltpu.VMEM((CHUNK, V), jnp.float32),
        pltpu.VMEM((CHUNK, V), jnp.float32),
        pltpu.VMEM((CHUNK, V), jnp.float32),
        pltpu.VMEM((CHUNK, V), jnp.float32),
        pltpu.SemaphoreType.DMA,
        pltpu.SemaphoreType.DMA,
        pltpu.SemaphoreType.DMA,
        pltpu.SemaphoreType.DMA,
        pltpu.SemaphoreType.DMA,
        pltpu.SemaphoreType.DMA,
    ],
    compiler_params=pltpu.CompilerParams(needs_layout_passes=False),
)
def _sc_onehot(x_hbm, out_hbm, idx_v, buf0, buf1, buf2, buf3, buf4, buf5, sem0, sem1, sem2, sem3, sem4, sem5):
    wid = lax.axis_index("s") * NC + lax.axis_index("c")
    base_row = wid * ROWS_PER_W

    pltpu.sync_copy(x_hbm.at[pl.ds(base_row, ROWS_PER_W)], idx_v)

    zeros16 = jnp.zeros((L,), jnp.float32)
    lane = lax.iota(jnp.int32, L)

    def _zero_row(r, carry):
        def _zero_k(k, c2):
            off = pl.multiple_of(k * L, L)
            buf0[r, pl.ds(off, L)] = zeros16
            buf1[r, pl.ds(off, L)] = zeros16
            buf2[r, pl.ds(off, L)] = zeros16
            buf3[r, pl.ds(off, L)] = zeros16
            buf4[r, pl.ds(off, L)] = zeros16
            buf5[r, pl.ds(off, L)] = zeros16
            return c2

        lax.fori_loop(0, V // L, _zero_k, 0)
        rows = jnp.full((L,), r, jnp.int32)
        tail = (V - L) + lane
        plsc.store_scatter(buf0, [rows, tail], zeros16)
        plsc.store_scatter(buf1, [rows, tail], zeros16)
        plsc.store_scatter(buf2, [rows, tail], zeros16)
        plsc.store_scatter(buf3, [rows, tail], zeros16)
        plsc.store_scatter(buf4, [rows, tail], zeros16)
        plsc.store_scatter(buf5, [rows, tail], zeros16)
        return carry

    lax.fori_loop(0, CHUNK, _zero_row, 0)

    bufs = (buf0, buf1, buf2, buf3, buf4, buf5)
    sems = (sem0, sem1, sem2, sem3, sem4, sem5)
    ones16 = jnp.ones((L,), jnp.float32)

    copies = [None] * NCHUNK
    positions = [None] * NCHUNK
    for c in range(NCHUNK):
        b = c % 6
        if c >= 6:
            copies[c - 6].wait()
            for rows, cols in positions[c - 6]:
                plsc.store_scatter(bufs[b], [rows, cols], zeros16)
        pos_list = []
        for g in range(GROUPS):
            xs = idx_v[pl.ds(c * CHUNK + g * L, L)]
            rows = g * L + lane
            plsc.store_scatter(bufs[b], [rows, xs], ones16)
            pos_list.append((rows, xs))
        positions[c] = pos_list
        row0 = base_row + c * CHUNK
        cp = pltpu.make_async_copy(
            bufs[b],
            out_hbm.at[pl.ds(row0, CHUNK), :],
            sems[b],
        )
        cp.start()
        copies[c] = cp
    for d in range(6):
        copies[NCHUNK - 6 + d].wait()


def kernel(X):
    return _sc_onehot(X)
